# Initial kernel scaffold; baseline (speedup 1.0000x reference)
import jax
import jax.numpy as jnp
from jax.experimental import pallas as pl


def kernel(in_tensor, time, bbox, sp0, sp1, sp2, tp0, tp1, tp2):
    raise NotImplementedError("write your pallas kernel here")


# speedup vs baseline: 46.1407x; 46.1407x over previous
import functools

import jax
import jax.numpy as jnp
from jax import lax
from jax.experimental import pallas as pl
from jax.experimental.pallas import tpu as pltpu
from jax.experimental.pallas import tpu_sc as plsc

_SPATIAL = (64, 128, 256)
_TEMPORAL = (32, 64, 128)
_C = 32
_NC, _NS, _L = 2, 16, 16
_NW = _NC * _NS
_P = 16

_BASES = []
_base = 0
for _l in range(3):
    _S, _T = _SPATIAL[_l], _TEMPORAL[_l]
    _BASES.append((_base, _base + 3 * _S * _S))
    _base += 3 * _S * _S + 3 * _T * _S
_R = _base


def _shift_w(a):
    return jnp.concatenate([a[..., 1:], a[..., -1:]], axis=-1)


def _shift_h(a):
    return jnp.concatenate([a[..., 1:, :], a[..., -1:, :]], axis=-2)


def _pack_group(arr):
    p10 = _shift_h(arr)
    st = jnp.stack([arr, _shift_w(arr), p10, _shift_w(p10)], axis=1)
    st = st.transpose(0, 3, 4, 1, 2)
    return st.reshape(-1, 4 * _C)


def _pack_table(sp0, sp1, sp2, tp0, tp1, tp2):
    groups = []
    for sp, tp in ((sp0, tp0), (sp1, tp1), (sp2, tp2)):
        groups.append(_pack_group(sp))
        groups.append(_pack_group(tp))
    return jnp.concatenate(groups, axis=0)


_GATHER_DNUMS = lax.GatherDimensionNumbers(
    offset_dims=(), collapsed_slice_dims=(0,), start_index_map=(0,))


def _bcast_lane(vec, lane_vec):
    return lax.gather(vec, lane_vec[:, None], _GATHER_DNUMS, (1,),
                      mode=lax.GatherScatterMode.PROMISE_IN_BOUNDS)


@functools.lru_cache(maxsize=None)
def _make_sc(N):
    NPW = N // _NW
    NB = NPW // _P
    mesh = plsc.VectorSubcoreMesh(core_axis_name="c", subcore_axis_name="s")

    @functools.partial(
        pl.kernel,
        out_type=jax.ShapeDtypeStruct((N * 96,), jnp.float32),
        mesh=mesh,
        scratch_types=[
            pltpu.VMEM((16,), jnp.float32),
            pltpu.VMEM((_P,), jnp.float32),
            pltpu.VMEM((_P,), jnp.float32),
            pltpu.VMEM((_P,), jnp.float32),
            pltpu.VMEM((_P,), jnp.float32),
            pltpu.VMEM((18, _P), jnp.int32),
            pltpu.VMEM((18 * 4, _P), jnp.float32),
            pltpu.VMEM((18, _P, 4 * _C), jnp.float32),
            pltpu.VMEM((_P * 96,), jnp.float32),
            pltpu.SemaphoreType.DMA,
        ],
    )
    def sc_fn(table, xs, ys, zs, ts, par, out_hbm,
              pbuf, xb, yb, zb, tb, idx_v, w_v, rows_v, ob, sem):
        wid = lax.axis_index("s") * _NC + lax.axis_index("c")
        pltpu.sync_copy(par, pbuf)
        pv = pbuf[pl.ds(0, 16)]
        lox, loy, loz = pv[0], pv[1], pv[2]
        ivx, ivy, ivz = pv[3], pv[4], pv[5]
        base_w = wid * NPW

        @pl.loop(0, NB)
        def _block(b):
            bp = base_w + b * _P
            pltpu.sync_copy(xs.at[pl.ds(bp, _P)], xb)
            pltpu.sync_copy(ys.at[pl.ds(bp, _P)], yb)
            pltpu.sync_copy(zs.at[pl.ds(bp, _P)], zb)
            pltpu.sync_copy(ts.at[pl.ds(bp, _P)], tb)

            sl = pl.ds(0, _L)
            ux = jnp.clip((xb[sl] - lox) * ivx, 0.0, 1.0)
            uy = jnp.clip((yb[sl] - loy) * ivy, 0.0, 1.0)
            uz = jnp.clip((zb[sl] - loz) * ivz, 0.0, 1.0)
            ut = tb[sl]
            for l in range(3):
                S, T = _SPATIAL[l], _TEMPORAL[l]
                spb, tpb = _BASES[l]
                px = ux * (S - 1.0)
                py = uy * (S - 1.0)
                pz = uz * (S - 1.0)
                pt = ut * (T - 1.0)
                ix = px.astype(jnp.int32)
                iy = py.astype(jnp.int32)
                iz = pz.astype(jnp.int32)
                it = pt.astype(jnp.int32)
                fx = px - ix.astype(jnp.float32)
                fy = py - iy.astype(jnp.float32)
                fz = pz - iz.astype(jnp.float32)
                ft = pt - it.astype(jnp.float32)
                gx, gy, gz, gt = 1.0 - fx, 1.0 - fy, 1.0 - fz, 1.0 - ft
                planes = (
                    (iy, ix, gy, fy, gx, fx, spb),
                    (iz, iy, gz, fz, gy, fy, spb + S * S),
                    (iz, ix, gz, fz, gx, fx, spb + 2 * S * S),
                    (it, ix, gt, ft, gx, fx, tpb),
                    (it, iy, gt, ft, gy, fy, tpb + T * S),
                    (it, iz, gt, ft, gz, fz, tpb + 2 * T * S),
                )
                for jj, (ih, iw, wh0, wh1, ww0, ww1, pb) in enumerate(planes):
                    j = l * 6 + jj
                    idx_v[j, sl] = ih * S + iw + pb
                    w_v[j * 4 + 0, sl] = wh0 * ww0
                    w_v[j * 4 + 1, sl] = wh0 * ww1
                    w_v[j * 4 + 2, sl] = wh1 * ww0
                    w_v[j * 4 + 3, sl] = wh1 * ww1

            descs = [
                pltpu.async_copy(table.at[idx_v.at[j]], rows_v.at[j], sem)
                for j in range(18)
            ]
            for d in descs:
                d.wait()

            @pl.loop(0, _P)
            def _pt(p):
                pvec = jnp.full((16,), p, jnp.int32)
                for l in range(3):
                    acc0 = None
                    acc1 = None
                    for jj in range(6):
                        j = l * 6 + jj
                        w00 = _bcast_lane(w_v[j * 4 + 0, sl], pvec)
                        w01 = _bcast_lane(w_v[j * 4 + 1, sl], pvec)
                        w10 = _bcast_lane(w_v[j * 4 + 2, sl], pvec)
                        w11 = _bcast_lane(w_v[j * 4 + 3, sl], pvec)
                        e0 = (rows_v[j, p, pl.ds(0, 16)] * w00
                              + rows_v[j, p, pl.ds(32, 16)] * w01
                              + rows_v[j, p, pl.ds(64, 16)] * w10
                              + rows_v[j, p, pl.ds(96, 16)] * w11)
                        e1 = (rows_v[j, p, pl.ds(16, 16)] * w00
                              + rows_v[j, p, pl.ds(48, 16)] * w01
                              + rows_v[j, p, pl.ds(80, 16)] * w10
                              + rows_v[j, p, pl.ds(112, 16)] * w11)
                        if acc0 is None:
                            acc0, acc1 = e0, e1
                        else:
                            acc0 = acc0 * e0
                            acc1 = acc1 * e1
                    ob[pl.ds(p * 96 + l * 32, 16)] = acc0
                    ob[pl.ds(p * 96 + l * 32 + 16, 16)] = acc1

            pltpu.sync_copy(ob, out_hbm.at[pl.ds(bp * 96, _P * 96)])

    return sc_fn


def kernel(in_tensor, time, bbox, sp0, sp1, sp2, tp0, tp1, tp2):
    sh = in_tensor.shape
    N = sh[0] * sh[1]
    pts = in_tensor.reshape(-1, 3)
    xs = pts[:, 0]
    ys = pts[:, 1]
    zs = pts[:, 2]
    ts = time.reshape(-1)
    lo = bbox[0]
    inv = 1.0 / (bbox[1] - bbox[0])
    par = jnp.concatenate([lo, inv, jnp.zeros(10, jnp.float32)])
    table = _pack_table(sp0, sp1, sp2, tp0, tp1, tp2)
    out = _make_sc(N)(table, xs, ys, zs, ts, par)
    return out.reshape(sh[0], sh[1], 96)
